# Initial kernel scaffold; baseline (speedup 1.0000x reference)
import jax
import jax.numpy as jnp
from jax.experimental import pallas as pl


def kernel(hidden_states, top_k_index, top_k_weights, gate_up_proj, down_proj):
    raise NotImplementedError("write your pallas kernel here")


# speedup vs baseline: 3.1525x; 3.1525x over previous
import functools

import jax
import jax.numpy as jnp
from jax import lax
from jax.experimental import pallas as pl
from jax.experimental.pallas import tpu as pltpu
from jax.experimental.pallas import tpu_sc as plsc

NUM_EXPERTS = 64
HIDDEN = 1024
INTER = 768
TOKENS = 2048
TOP_K = 2
PAIRS = TOKENS * TOP_K
ALIGN = 8
RT = 64
PBUF = 4608

SC_CORES = 2
SC_SUBCORES = 16
SC_WORKERS = SC_CORES * SC_SUBCORES

A_PER_W = PBUF // SC_WORKERS
A_CHUNK = 72
A_NCHUNK = A_PER_W // A_CHUNK

C_PER_W = TOKENS // SC_WORKERS
C_CHUNK = 32
C_NCHUNK = C_PER_W // C_CHUNK

def _worker_id():
    return lax.axis_index("s") * SC_CORES + lax.axis_index("c")


@functools.lru_cache(maxsize=None)
def _sc_kernels():
    mesh = plsc.VectorSubcoreMesh(core_axis_name="c", subcore_axis_name="s")

    @functools.partial(
        pl.kernel,
        mesh=mesh,
        out_type=jax.ShapeDtypeStruct((PBUF, HIDDEN), jnp.float32),
        scratch_types=[
            pltpu.VMEM((A_CHUNK,), jnp.int32),
            pltpu.VMEM((A_CHUNK, HIDDEN), jnp.float32),
            pltpu.SemaphoreType.DMA,
        ],
    )
    def sc_gather(hidden_hbm, srctok_hbm, out_hbm, idx_v, rows_v, sem):
        wid = _worker_id()
        for c in range(A_NCHUNK):
            base = wid * A_PER_W + c * A_CHUNK
            pltpu.sync_copy(srctok_hbm.at[pl.ds(base, A_CHUNK)], idx_v)
            pltpu.async_copy(hidden_hbm.at[idx_v], rows_v, sem).wait()
            pltpu.sync_copy(rows_v, out_hbm.at[pl.ds(base, A_CHUNK)])

    @functools.partial(
        pl.kernel,
        mesh=mesh,
        out_type=jax.ShapeDtypeStruct((TOKENS, HIDDEN), jnp.float32),
        scratch_types=[
            pltpu.VMEM((C_CHUNK,), jnp.int32),
            pltpu.VMEM((C_CHUNK,), jnp.int32),
            pltpu.VMEM((C_CHUNK,), jnp.float32),
            pltpu.VMEM((C_CHUNK,), jnp.float32),
            pltpu.VMEM((C_CHUNK, HIDDEN), jnp.float32),
            pltpu.VMEM((C_CHUNK, HIDDEN), jnp.float32),
            pltpu.SemaphoreType.DMA,
            pltpu.SemaphoreType.DMA,
        ],
    )
    def sc_combine(ypad_hbm, i0_hbm, i1_hbm, w0_hbm, w1_hbm, out_hbm,
                   ia_v, ib_v, wa_v, wb_v, a_v, b_v, sem_a, sem_b):
        wid = _worker_id()
        for c in range(C_NCHUNK):
            base = wid * C_PER_W + c * C_CHUNK
            pltpu.sync_copy(i0_hbm.at[pl.ds(base, C_CHUNK)], ia_v)
            pltpu.sync_copy(i1_hbm.at[pl.ds(base, C_CHUNK)], ib_v)
            pltpu.sync_copy(w0_hbm.at[pl.ds(base, C_CHUNK)], wa_v)
            pltpu.sync_copy(w1_hbm.at[pl.ds(base, C_CHUNK)], wb_v)
            cp_a = pltpu.async_copy(ypad_hbm.at[ia_v], a_v, sem_a)
            cp_b = pltpu.async_copy(ypad_hbm.at[ib_v], b_v, sem_b)
            cp_a.wait()
            cp_b.wait()

            for r in range(C_CHUNK):
                wa = wa_v[pl.ds((r // 16) * 16, 16)][r % 16]
                wb = wb_v[pl.ds((r // 16) * 16, 16)][r % 16]

                def col_body(j, _, r=r, wa=wa, wb=wb):
                    sl = pl.ds(j * 16, 16)
                    a_v[r, sl] = a_v[r, sl] * wa + b_v[r, sl] * wb
                    return 0
                lax.fori_loop(0, HIDDEN // 16, col_body, 0, unroll=8)
            pltpu.sync_copy(a_v, out_hbm.at[pl.ds(base, C_CHUNK)])

    return sc_gather, sc_combine


def _tc_moe_body(poff_ref, x_ref, gu_ref, dn_ref, y_ref):
    e = pl.program_id(0)
    start = poff_ref[e]
    end = poff_ref[e + 1]
    ntiles = (end - start + RT - 1) // RT
    dn = dn_ref[0]

    def tile(i, _):
        r0 = pl.multiple_of(start + i * RT, ALIGN)
        x = x_ref[pl.ds(r0, RT), :]
        g = lax.dot_general(x, gu_ref[0, :INTER, :],
                            (((1,), (1,)), ((), ())),
                            preferred_element_type=jnp.float32)
        u = lax.dot_general(x, gu_ref[0, INTER:, :],
                            (((1,), (1,)), ((), ())),
                            preferred_element_type=jnp.float32)
        h = g * jax.nn.sigmoid(g) * u
        y = lax.dot_general(h, dn, (((1,), (1,)), ((), ())),
                            preferred_element_type=jnp.float32)
        y_ref[pl.ds(r0, RT), :] = y
        return 0

    lax.fori_loop(0, ntiles, tile, 0)


def _tc_moe(x_pad, gate_up_proj, down_proj, poff, interpret=False):
    grid_spec = pltpu.PrefetchScalarGridSpec(
        num_scalar_prefetch=1,
        grid=(NUM_EXPERTS,),
        in_specs=[
            pl.BlockSpec((PBUF, HIDDEN), lambda e, poff: (0, 0)),
            pl.BlockSpec((1, 2 * INTER, HIDDEN), lambda e, poff: (e, 0, 0)),
            pl.BlockSpec((1, HIDDEN, INTER), lambda e, poff: (e, 0, 0)),
        ],
        out_specs=pl.BlockSpec((PBUF, HIDDEN), lambda e, poff: (0, 0)),
    )
    return pl.pallas_call(
        _tc_moe_body,
        grid_spec=grid_spec,
        out_shape=jax.ShapeDtypeStruct((PBUF, HIDDEN), jnp.float32),
        compiler_params=pltpu.CompilerParams(
            dimension_semantics=("arbitrary",),
        ),
        interpret=interpret,
    )(poff, x_pad, gate_up_proj, down_proj)


def _routing(top_k_index, top_k_weights):
    e_flat = top_k_index.astype(jnp.int32).reshape(-1)
    order = jnp.argsort(e_flat).astype(jnp.int32)
    e_sorted = e_flat[order]
    counts = jnp.bincount(e_flat, length=NUM_EXPERTS).astype(jnp.int32)
    cnt_pad = (counts + (ALIGN - 1)) // ALIGN * ALIGN
    zero = jnp.zeros((1,), jnp.int32)
    poff = jnp.concatenate([zero, jnp.cumsum(cnt_pad).astype(jnp.int32)])
    off = jnp.concatenate([zero, jnp.cumsum(counts).astype(jnp.int32)])
    rank = jnp.arange(PAIRS, dtype=jnp.int32) - off[e_sorted]
    ppos = poff[e_sorted] + rank
    tok_sorted = (order // TOP_K).astype(jnp.int32)
    src_tok = jnp.zeros((PBUF,), jnp.int32).at[ppos].set(tok_sorted)
    inv = jnp.zeros((PAIRS,), jnp.int32).at[order].set(ppos)
    inv = inv.reshape(TOKENS, TOP_K)
    i0 = inv[:, 0]
    i1 = inv[:, 1]
    return src_tok, poff, i0, i1


def kernel(hidden_states, top_k_index, top_k_weights, gate_up_proj, down_proj):
    src_tok, poff, i0, i1 = _routing(top_k_index, top_k_weights)
    w = top_k_weights.astype(jnp.float32)
    sc_gather, sc_combine = _sc_kernels()
    x_pad = sc_gather(hidden_states.astype(jnp.float32), src_tok)
    y_pad = _tc_moe(x_pad, gate_up_proj, down_proj, poff)
    return sc_combine(y_pad, i0, i1, w[:, 0], w[:, 1])
